# baseline (speedup 1.0000x reference)
import functools

import jax
import jax.numpy as jnp
from jax import lax
from jax.experimental import pallas as pl
from jax.experimental.pallas import tpu as pltpu
from jax.experimental.pallas import tpu_sc as plsc

H = 128
NW = 32
LANES = 16


def _round_up(x, m):
    return -(-x // m) * m


def _make_g1(n_bonds, atoms_p, nb):
    apw = atoms_p // NW
    ca = 128 // nb
    nch = apw // 8
    sub = 8 // ca
    rows_pw = atoms_p * nb // 128 // NW

    @functools.partial(
        pl.kernel,
        mesh=plsc.VectorSubcoreMesh(core_axis_name="c", subcore_axis_name="s"),
        out_type=jax.ShapeDtypeStruct((atoms_p, H), jnp.float32),
        scratch_types=[
            pltpu.VMEM((rows_pw, 128), jnp.int32),
            pltpu.VMEM((128, H), jnp.float32),
            pltpu.VMEM((8, H), jnp.float32),
            pltpu.SemaphoreType.DMA,
        ],
    )
    def g1(msg_hbm, a2b_hbm, out_hbm, idx_v, rows_v, acc_v, sem):
        wid = lax.axis_index("s") * 2 + lax.axis_index("c")
        pltpu.sync_copy(a2b_hbm.at[pl.ds(wid * rows_pw, rows_pw)], idx_v)
        abase = wid * apw

        def chunk(c, carry):
            for h in range(sub):
                pltpu.async_copy(
                    msg_hbm.at[idx_v.at[c * sub + h]], rows_v, sem).wait()
                for a in range(ca):
                    for g in range(H // LANES):
                        sl = pl.ds(g * LANES, LANES)
                        acc = rows_v[a * nb, sl]
                        for k in range(1, nb):
                            acc = acc + rows_v[a * nb + k, sl]
                        acc_v[h * ca + a, sl] = acc
            pltpu.sync_copy(acc_v, out_hbm.at[pl.ds(abase + c * 8, 8)])
            return carry

        lax.fori_loop(0, nch, chunk, 0)

    return g1


def _make_g2(bonds_p):
    bpw = bonds_p // NW
    nch = bpw // 128
    rows_pw = bonds_p // 128 // NW

    @functools.partial(
        pl.kernel,
        mesh=plsc.VectorSubcoreMesh(core_axis_name="c", subcore_axis_name="s"),
        out_type=jax.ShapeDtypeStruct((bonds_p, H), jnp.float32),
        scratch_types=[
            pltpu.VMEM((rows_pw, 128), jnp.int32),
            pltpu.VMEM((rows_pw, 128), jnp.int32),
            pltpu.VMEM((128, H), jnp.float32),
            pltpu.VMEM((128, H), jnp.float32),
            pltpu.SemaphoreType.DMA,
            pltpu.SemaphoreType.DMA,
        ],
    )
    def g2(am_hbm, msg_hbm, b2a_hbm, b2revb_hbm, out_hbm,
           idxa_v, idxb_v, bufa_v, bufb_v, sema, semb):
        wid = lax.axis_index("s") * 2 + lax.axis_index("c")
        pltpu.sync_copy(b2a_hbm.at[pl.ds(wid * rows_pw, rows_pw)], idxa_v)
        pltpu.sync_copy(b2revb_hbm.at[pl.ds(wid * rows_pw, rows_pw)], idxb_v)
        bbase = wid * bpw

        def chunk(c, carry):
            ha = pltpu.async_copy(am_hbm.at[idxa_v.at[c]], bufa_v, sema)
            hb = pltpu.async_copy(msg_hbm.at[idxb_v.at[c]], bufb_v, semb)
            ha.wait()
            hb.wait()
            for r in range(128):
                for g in range(H // LANES):
                    sl = pl.ds(g * LANES, LANES)
                    bufa_v[r, sl] = bufa_v[r, sl] - bufb_v[r, sl]
            pltpu.sync_copy(bufa_v, out_hbm.at[pl.ds(bbase + c * 128, 128)])
            return carry

        lax.fori_loop(0, nch, chunk, 0)

    return g2


def _m1(f_bonds, W_i):
    n, fd = f_bonds.shape
    blk = 512
    grid = n // blk

    def body(x_ref, w_ref, inp_ref, msg_ref):
        y = jnp.dot(x_ref[...], w_ref[...], preferred_element_type=jnp.float32)
        inp_ref[...] = y
        msg_ref[...] = jnp.maximum(y, 0.0)

    return pl.pallas_call(
        body,
        grid=(grid,),
        in_specs=[pl.BlockSpec((blk, fd), lambda i: (i, 0)),
                  pl.BlockSpec((fd, H), lambda i: (0, 0))],
        out_specs=[pl.BlockSpec((blk, H), lambda i: (i, 0)),
                   pl.BlockSpec((blk, H), lambda i: (i, 0))],
        out_shape=[jax.ShapeDtypeStruct((n, H), jnp.float32),
                   jax.ShapeDtypeStruct((n, H), jnp.float32)],
    )(f_bonds, W_i)


def _m3(inp, pre, W_h):
    n = inp.shape[0]
    blk = 512
    grid = n // blk

    def body(i_ref, p_ref, w_ref, o_ref):
        y = i_ref[...] + jnp.dot(p_ref[...], w_ref[...],
                                 preferred_element_type=jnp.float32)
        o_ref[...] = jnp.maximum(y, 0.0)

    return pl.pallas_call(
        body,
        grid=(grid,),
        in_specs=[pl.BlockSpec((blk, H), lambda i: (i, 0)),
                  pl.BlockSpec((blk, H), lambda i: (i, 0)),
                  pl.BlockSpec((H, H), lambda i: (0, 0))],
        out_specs=pl.BlockSpec((blk, H), lambda i: (i, 0)),
        out_shape=jax.ShapeDtypeStruct((n, H), jnp.float32),
    )(inp, pre, W_h)


def _m4(f_atoms_p, am_p, mol3d, W_o, b_o2d, n_mols):
    atoms_p, afd = f_atoms_p.shape
    blk = 512
    grid = atoms_p // blk

    def body(fa_ref, am_ref, id_ref, w_ref, b_ref, out_ref, cnt_ref):
        i = pl.program_id(0)

        @pl.when(i == 0)
        def _init():
            out_ref[...] = jnp.zeros_like(out_ref)
            cnt_ref[...] = jnp.zeros_like(cnt_ref)

        hid = (jnp.dot(fa_ref[...], w_ref[:afd, :],
                       preferred_element_type=jnp.float32)
               + jnp.dot(am_ref[...], w_ref[afd:, :],
                         preferred_element_type=jnp.float32)
               + b_ref[...])
        hid = jnp.maximum(hid, 0.0)
        ids = id_ref[0, 0, :]
        onehot = (ids[:, None]
                  == lax.broadcasted_iota(jnp.int32, (blk, n_mols), 1)
                  ).astype(jnp.float32)
        out_ref[...] += lax.dot_general(
            onehot, hid, (((0,), (0,)), ((), ())),
            preferred_element_type=jnp.float32)
        cnt_ref[...] = cnt_ref[...] + jnp.sum(onehot, axis=0)[:, None]

        @pl.when(i == grid - 1)
        def _fini():
            out_ref[...] = out_ref[...] / jnp.maximum(cnt_ref[...], 1.0)

    return pl.pallas_call(
        body,
        grid=(grid,),
        in_specs=[pl.BlockSpec((blk, afd), lambda i: (i, 0)),
                  pl.BlockSpec((blk, H), lambda i: (i, 0)),
                  pl.BlockSpec((1, 1, blk), lambda i: (i, 0, 0)),
                  pl.BlockSpec((afd + H, H), lambda i: (0, 0)),
                  pl.BlockSpec((1, H), lambda i: (0, 0))],
        out_specs=pl.BlockSpec((n_mols, H), lambda i: (0, 0)),
        out_shape=jax.ShapeDtypeStruct((n_mols, H), jnp.float32),
        scratch_shapes=[pltpu.VMEM((n_mols, H), jnp.float32)],
    )(f_atoms_p, am_p, mol3d, W_o, b_o2d)


def _encode(f_atoms, f_bonds, a2b, b2a, b2revb, mol_ids,
            W_i, W_h, W_o, b_o2d, depth, n_mols):
    n_atoms, nb = a2b.shape
    n_bonds = f_bonds.shape[0]
    atoms_p = _round_up(n_atoms, 2560)
    bonds_p = _round_up(n_bonds, 32768)

    a2b2d = jnp.pad(a2b.astype(jnp.int32),
                    ((0, atoms_p - n_atoms), (0, 0))).reshape(-1, 128)
    b2a2d = jnp.pad(b2a.astype(jnp.int32),
                    (0, bonds_p - n_bonds)).reshape(-1, 128)
    b2revb2d = jnp.pad(b2revb.astype(jnp.int32),
                       (0, bonds_p - n_bonds)).reshape(-1, 128)
    f_atoms_p = jnp.pad(f_atoms, ((0, atoms_p - n_atoms), (0, 0)))
    mol3d = jnp.pad(mol_ids.astype(jnp.int32), (0, atoms_p - n_atoms),
                    constant_values=n_mols).reshape(atoms_p // 512, 1, 512)

    g1 = _make_g1(n_bonds, atoms_p, nb)
    g2 = _make_g2(bonds_p)

    inp, msg = _m1(f_bonds, W_i)
    for _ in range(depth - 1):
        am = g1(msg, a2b2d)
        pre = g2(am, msg, b2a2d, b2revb2d)
        msg = _m3(inp, pre, W_h)
    am = g1(msg, a2b2d)
    return _m4(f_atoms_p, am, mol3d, W_o, b_o2d, n_mols)


def kernel(f_atoms, f_bonds, a2b, b2a, b2revb, atom_mol_ids,
           ano_f_atoms, ano_f_bonds, ano_a2b, ano_b2a, ano_b2revb,
           ano_atom_mol_ids, W_i, W_h, W_o, b_o):
    depth = 3
    n_mols = 256
    b_o2d = b_o.reshape(1, H)
    mol_vecs = _encode(f_atoms, f_bonds, a2b, b2a, b2revb, atom_mol_ids,
                       W_i, W_h, W_o, b_o2d, depth, n_mols)
    ano_mol_vecs = _encode(ano_f_atoms, ano_f_bonds, ano_a2b, ano_b2a,
                           ano_b2revb, ano_atom_mol_ids,
                           W_i, W_h, W_o, b_o2d, depth, n_mols)
    return (mol_vecs, ano_mol_vecs)


# speedup vs baseline: 1.2485x; 1.2485x over previous
import functools

import jax
import jax.numpy as jnp
from jax import lax
from jax.experimental import pallas as pl
from jax.experimental.pallas import tpu as pltpu
from jax.experimental.pallas import tpu_sc as plsc

H = 128
NW = 32
LANES = 16


def _round_up(x, m):
    return -(-x // m) * m


def _make_g1(n_bonds, atoms_p, nb):
    apw = atoms_p // NW
    ca = 128 // nb
    nch = apw // ca
    rows_pw = atoms_p * nb // 128 // NW
    nbuf = 4
    assert nch % nbuf == 0 and rows_pw == nch

    @functools.partial(
        pl.kernel,
        mesh=plsc.VectorSubcoreMesh(core_axis_name="c", subcore_axis_name="s"),
        out_type=jax.ShapeDtypeStruct((atoms_p, H), jnp.float32),
        scratch_types=[
            pltpu.VMEM((rows_pw, 128), jnp.int32),
            pltpu.VMEM((nbuf * 128, H), jnp.float32),
            pltpu.VMEM((apw, H), jnp.float32),
            pltpu.SemaphoreType.DMA,
            pltpu.SemaphoreType.DMA,
            pltpu.SemaphoreType.DMA,
            pltpu.SemaphoreType.DMA,
        ],
    )
    def g1(msg_hbm, a2b_hbm, out_hbm, idx_v, rows_v, acc_v, *sems):
        wid = lax.axis_index("s") * 2 + lax.axis_index("c")
        pltpu.sync_copy(a2b_hbm.at[pl.ds(wid * rows_pw, rows_pw)], idx_v)
        abase = wid * apw

        def rows_slot(b):
            return rows_v.at[pl.ds(b * 128, 128), :]

        def start(c, b):
            pltpu.async_copy(msg_hbm.at[idx_v.at[c]], rows_slot(b), sems[b])

        def wait(b):
            pltpu.make_async_copy(
                msg_hbm.at[idx_v.at[0]], rows_slot(b), sems[b]).wait()

        for b in range(nbuf):
            start(b, b)

        def outer(j, carry):
            cc = j * nbuf
            for b in range(nbuf):
                c = cc + b
                wait(b)

                def acc_a(a, carry2):
                    base = b * 128 + a * nb
                    for g in range(H // LANES):
                        sl = pl.ds(g * LANES, LANES)
                        v = rows_v[base, sl]
                        for k in range(1, nb):
                            v = v + rows_v[base + k, sl]
                        acc_v[c * ca + a, sl] = v
                    return carry2

                lax.fori_loop(0, ca, acc_a, 0)

                @pl.when(c + nbuf < nch)
                def _next():
                    start(c + nbuf, b)
            return carry

        lax.fori_loop(0, nch // nbuf, outer, 0)
        pltpu.sync_copy(acc_v, out_hbm.at[pl.ds(abase, apw)])

    return g1


def _make_g2(bonds_p):
    bpw = bonds_p // NW
    nch = bpw // 128
    rows_pw = bonds_p // 128 // NW
    nbuf = 2
    assert nch % nbuf == 0 and rows_pw == nch

    @functools.partial(
        pl.kernel,
        mesh=plsc.VectorSubcoreMesh(core_axis_name="c", subcore_axis_name="s"),
        out_type=jax.ShapeDtypeStruct((bonds_p, H), jnp.float32),
        scratch_types=[
            pltpu.VMEM((rows_pw, 128), jnp.int32),
            pltpu.VMEM((rows_pw, 128), jnp.int32),
            pltpu.VMEM((nbuf * 128, H), jnp.float32),
            pltpu.VMEM((nbuf * 128, H), jnp.float32),
            pltpu.VMEM((nbuf * 128, H), jnp.float32),
            pltpu.SemaphoreType.DMA,
            pltpu.SemaphoreType.DMA,
            pltpu.SemaphoreType.DMA,
            pltpu.SemaphoreType.DMA,
            pltpu.SemaphoreType.DMA,
            pltpu.SemaphoreType.DMA,
        ],
    )
    def g2(am_hbm, msg_hbm, b2a_hbm, b2revb_hbm, out_hbm,
           idxa_v, idxb_v, bufa_v, bufb_v, outb_v, *sems):
        sema = sems[0:nbuf]
        semb = sems[nbuf:2 * nbuf]
        semo = sems[2 * nbuf:3 * nbuf]
        wid = lax.axis_index("s") * 2 + lax.axis_index("c")
        pltpu.sync_copy(b2a_hbm.at[pl.ds(wid * rows_pw, rows_pw)], idxa_v)
        pltpu.sync_copy(b2revb_hbm.at[pl.ds(wid * rows_pw, rows_pw)], idxb_v)
        bbase = wid * bpw

        def slot(ref, b):
            return ref.at[pl.ds(b * 128, 128), :]

        def start(c, b):
            pltpu.async_copy(am_hbm.at[idxa_v.at[c]], slot(bufa_v, b),
                             sema[b])
            pltpu.async_copy(msg_hbm.at[idxb_v.at[c]], slot(bufb_v, b),
                             semb[b])

        def wait_in(b):
            pltpu.make_async_copy(
                am_hbm.at[idxa_v.at[0]], slot(bufa_v, b), sema[b]).wait()
            pltpu.make_async_copy(
                msg_hbm.at[idxb_v.at[0]], slot(bufb_v, b), semb[b]).wait()

        def wait_out(b):
            pltpu.make_async_copy(
                slot(outb_v, b), out_hbm.at[pl.ds(bbase, 128)],
                semo[b]).wait()

        for b in range(nbuf):
            start(b, b)

        def outer(j, carry):
            cc = j * nbuf
            for b in range(nbuf):
                c = cc + b

                @pl.when(cc > 0)
                def _drain():
                    wait_out(b)

                wait_in(b)

                def sub_r(r, carry2):
                    for g in range(H // LANES):
                        sl = pl.ds(g * LANES, LANES)
                        outb_v[b * 128 + r, sl] = (
                            bufa_v[b * 128 + r, sl] - bufb_v[b * 128 + r, sl])
                    return carry2

                lax.fori_loop(0, 128, sub_r, 0)
                pltpu.async_copy(
                    slot(outb_v, b), out_hbm.at[pl.ds(bbase + c * 128, 128)],
                    semo[b])

                @pl.when(c + nbuf < nch)
                def _next():
                    start(c + nbuf, b)
            return carry

        lax.fori_loop(0, nch // nbuf, outer, 0)
        for b in range(nbuf):
            wait_out(b)

    return g2


def _m1(f_bonds, W_i):
    n, fd = f_bonds.shape
    blk = 512
    grid = n // blk

    def body(x_ref, w_ref, inp_ref, msg_ref):
        y = jnp.dot(x_ref[...], w_ref[...], preferred_element_type=jnp.float32)
        inp_ref[...] = y
        msg_ref[...] = jnp.maximum(y, 0.0)

    return pl.pallas_call(
        body,
        grid=(grid,),
        in_specs=[pl.BlockSpec((blk, fd), lambda i: (i, 0)),
                  pl.BlockSpec((fd, H), lambda i: (0, 0))],
        out_specs=[pl.BlockSpec((blk, H), lambda i: (i, 0)),
                   pl.BlockSpec((blk, H), lambda i: (i, 0))],
        out_shape=[jax.ShapeDtypeStruct((n, H), jnp.float32),
                   jax.ShapeDtypeStruct((n, H), jnp.float32)],
    )(f_bonds, W_i)


def _m3(inp, pre, W_h):
    n = inp.shape[0]
    blk = 512
    grid = n // blk

    def body(i_ref, p_ref, w_ref, o_ref):
        y = i_ref[...] + jnp.dot(p_ref[...], w_ref[...],
                                 preferred_element_type=jnp.float32)
        o_ref[...] = jnp.maximum(y, 0.0)

    return pl.pallas_call(
        body,
        grid=(grid,),
        in_specs=[pl.BlockSpec((blk, H), lambda i: (i, 0)),
                  pl.BlockSpec((blk, H), lambda i: (i, 0)),
                  pl.BlockSpec((H, H), lambda i: (0, 0))],
        out_specs=pl.BlockSpec((blk, H), lambda i: (i, 0)),
        out_shape=jax.ShapeDtypeStruct((n, H), jnp.float32),
    )(inp, pre, W_h)


def _m4(f_atoms_p, am_p, mol3d, W_o, b_o2d, n_mols):
    atoms_p, afd = f_atoms_p.shape
    blk = 512
    grid = atoms_p // blk

    def body(fa_ref, am_ref, id_ref, w_ref, b_ref, out_ref, cnt_ref):
        i = pl.program_id(0)

        @pl.when(i == 0)
        def _init():
            out_ref[...] = jnp.zeros_like(out_ref)
            cnt_ref[...] = jnp.zeros_like(cnt_ref)

        hid = (jnp.dot(fa_ref[...], w_ref[:afd, :],
                       preferred_element_type=jnp.float32)
               + jnp.dot(am_ref[...], w_ref[afd:, :],
                         preferred_element_type=jnp.float32)
               + b_ref[...])
        hid = jnp.maximum(hid, 0.0)
        ids = id_ref[0, 0, :]
        onehot = (ids[:, None]
                  == lax.broadcasted_iota(jnp.int32, (blk, n_mols), 1)
                  ).astype(jnp.float32)
        out_ref[...] += lax.dot_general(
            onehot, hid, (((0,), (0,)), ((), ())),
            preferred_element_type=jnp.float32)
        cnt_ref[...] = cnt_ref[...] + jnp.sum(onehot, axis=0)[:, None]

        @pl.when(i == grid - 1)
        def _fini():
            out_ref[...] = out_ref[...] / jnp.maximum(cnt_ref[...], 1.0)

    return pl.pallas_call(
        body,
        grid=(grid,),
        in_specs=[pl.BlockSpec((blk, afd), lambda i: (i, 0)),
                  pl.BlockSpec((blk, H), lambda i: (i, 0)),
                  pl.BlockSpec((1, 1, blk), lambda i: (i, 0, 0)),
                  pl.BlockSpec((afd + H, H), lambda i: (0, 0)),
                  pl.BlockSpec((1, H), lambda i: (0, 0))],
        out_specs=pl.BlockSpec((n_mols, H), lambda i: (0, 0)),
        out_shape=jax.ShapeDtypeStruct((n_mols, H), jnp.float32),
        scratch_shapes=[pltpu.VMEM((n_mols, H), jnp.float32)],
    )(f_atoms_p, am_p, mol3d, W_o, b_o2d)


def _encode(f_atoms, f_bonds, a2b, b2a, b2revb, mol_ids,
            W_i, W_h, W_o, b_o2d, depth, n_mols):
    n_atoms, nb = a2b.shape
    n_bonds = f_bonds.shape[0]
    atoms_p = _round_up(n_atoms, 2560)
    bonds_p = _round_up(n_bonds, 32768)

    a2b2d = jnp.pad(a2b.astype(jnp.int32),
                    ((0, atoms_p - n_atoms), (0, 0))).reshape(-1, 128)
    b2a2d = jnp.pad(b2a.astype(jnp.int32),
                    (0, bonds_p - n_bonds)).reshape(-1, 128)
    b2revb2d = jnp.pad(b2revb.astype(jnp.int32),
                       (0, bonds_p - n_bonds)).reshape(-1, 128)
    f_atoms_p = jnp.pad(f_atoms, ((0, atoms_p - n_atoms), (0, 0)))
    mol3d = jnp.pad(mol_ids.astype(jnp.int32), (0, atoms_p - n_atoms),
                    constant_values=n_mols).reshape(atoms_p // 512, 1, 512)

    g1 = _make_g1(n_bonds, atoms_p, nb)
    g2 = _make_g2(bonds_p)

    inp, msg = _m1(f_bonds, W_i)
    for _ in range(depth - 1):
        am = g1(msg, a2b2d)
        pre = g2(am, msg, b2a2d, b2revb2d)
        msg = _m3(inp, pre, W_h)
    am = g1(msg, a2b2d)
    return _m4(f_atoms_p, am, mol3d, W_o, b_o2d, n_mols)


def kernel(f_atoms, f_bonds, a2b, b2a, b2revb, atom_mol_ids,
           ano_f_atoms, ano_f_bonds, ano_a2b, ano_b2a, ano_b2revb,
           ano_atom_mol_ids, W_i, W_h, W_o, b_o):
    depth = 3
    n_mols = 256
    b_o2d = b_o.reshape(1, H)
    mol_vecs = _encode(f_atoms, f_bonds, a2b, b2a, b2revb, atom_mol_ids,
                       W_i, W_h, W_o, b_o2d, depth, n_mols)
    ano_mol_vecs = _encode(ano_f_atoms, ano_f_bonds, ano_a2b, ano_b2a,
                           ano_b2revb, ano_atom_mol_ids,
                           W_i, W_h, W_o, b_o2d, depth, n_mols)
    return (mol_vecs, ano_mol_vecs)


# speedup vs baseline: 2.0625x; 1.6520x over previous
import functools

import jax
import jax.numpy as jnp
from jax import lax
from jax.experimental import pallas as pl
from jax.experimental.pallas import tpu as pltpu
from jax.experimental.pallas import tpu_sc as plsc

H = 128
NW = 32
LANES = 16


def _round_up(x, m):
    return -(-x // m) * m


def _make_g1(n_bonds, atoms_p, nb):
    apw = atoms_p // NW
    ck = 256
    ca = ck // nb
    nch = apw // ca
    idx_pw = apw * nb
    nbuf = 3
    nvisit = -(-nch // nbuf) * nbuf

    @functools.partial(
        pl.kernel,
        mesh=plsc.VectorSubcoreMesh(core_axis_name="c", subcore_axis_name="s"),
        out_type=jax.ShapeDtypeStruct((atoms_p, H), jnp.float32),
        scratch_types=[
            pltpu.VMEM((idx_pw,), jnp.int32),
            pltpu.VMEM((nbuf * ck, H), jnp.float32),
            pltpu.VMEM((nbuf * ca, H), jnp.float32),
            pltpu.SemaphoreType.DMA,
            pltpu.SemaphoreType.DMA,
            pltpu.SemaphoreType.DMA,
            pltpu.SemaphoreType.DMA,
            pltpu.SemaphoreType.DMA,
            pltpu.SemaphoreType.DMA,
        ],
    )
    def g1(msg_hbm, a2b_hbm, out_hbm, idx_v, rows_v, acc_v, *sems):
        semg = sems[0:nbuf]
        semo = sems[nbuf:2 * nbuf]
        wid = lax.axis_index("s") * 2 + lax.axis_index("c")
        pltpu.sync_copy(a2b_hbm.at[pl.ds(wid * idx_pw, idx_pw)], idx_v)
        abase = wid * apw

        def rows_slot(b):
            return rows_v.at[pl.ds(b * ck, ck), :]

        def acc_slot(b):
            return acc_v.at[pl.ds(b * ca, ca), :]

        def start(c, b):
            pltpu.async_copy(
                msg_hbm.at[idx_v.at[pl.ds(c * ck, ck)]], rows_slot(b),
                semg[b])

        def wait_in(b):
            pltpu.make_async_copy(
                msg_hbm.at[idx_v.at[pl.ds(0, ck)]], rows_slot(b),
                semg[b]).wait()

        def wait_out(b):
            pltpu.make_async_copy(
                acc_slot(b), out_hbm.at[pl.ds(abase, ca)], semo[b]).wait()

        for b in range(nbuf):
            start(b, b)

        def outer(j, carry):
            cc = j * nbuf
            for b in range(nbuf):
                c = cc + b

                @pl.when(c < nch)
                def _visit():
                    @pl.when(cc > 0)
                    def _drain():
                        wait_out(b)

                    wait_in(b)

                    def acc_a(a, carry2):
                        base = b * ck + a * nb
                        for g in range(H // LANES):
                            sl = pl.ds(g * LANES, LANES)
                            v = rows_v[base, sl]
                            for k in range(1, nb):
                                v = v + rows_v[base + k, sl]
                            acc_v[b * ca + a, sl] = v
                        return carry2

                    lax.fori_loop(0, ca, acc_a, 0)
                    pltpu.async_copy(
                        acc_slot(b), out_hbm.at[pl.ds(abase + c * ca, ca)],
                        semo[b])

                    @pl.when(c + nbuf < nch)
                    def _next():
                        start(c + nbuf, b)
            return carry

        lax.fori_loop(0, nvisit // nbuf, outer, 0)
        for b in range(nbuf):
            wait_out(b)

    return g1


def _make_g2(bonds_p, atoms_p):
    bpw = bonds_p // NW
    nch = bpw // 128
    rows_pw = bonds_p // 128 // NW
    nbuf = 2
    assert nch % nbuf == 0 and rows_pw == nch

    @functools.partial(
        pl.kernel,
        mesh=plsc.VectorSubcoreMesh(core_axis_name="c", subcore_axis_name="s"),
        out_type=jax.ShapeDtypeStruct((bonds_p, H), jnp.float32),
        scratch_types=[
            pltpu.VMEM((rows_pw, 128), jnp.int32),
            pltpu.VMEM((rows_pw, 128), jnp.int32),
            pltpu.VMEM((nbuf * 128, H), jnp.float32),
            pltpu.VMEM((nbuf * 128, H), jnp.float32),
            pltpu.VMEM((nbuf * 128, H), jnp.float32),
            pltpu.SemaphoreType.DMA,
            pltpu.SemaphoreType.DMA,
            pltpu.SemaphoreType.DMA,
            pltpu.SemaphoreType.DMA,
            pltpu.SemaphoreType.DMA,
            pltpu.SemaphoreType.DMA,
        ],
    )
    def g2(am_hbm, msg_hbm, b2a_hbm, b2revb_hbm, out_hbm,
           idxa_v, idxb_v, bufa_v, bufb_v, outb_v, *sems):
        sema = sems[0:nbuf]
        semb = sems[nbuf:2 * nbuf]
        semo = sems[2 * nbuf:3 * nbuf]
        wid = lax.axis_index("s") * 2 + lax.axis_index("c")
        pltpu.sync_copy(b2a_hbm.at[pl.ds(wid * rows_pw, rows_pw)], idxa_v)
        pltpu.sync_copy(b2revb_hbm.at[pl.ds(wid * rows_pw, rows_pw)], idxb_v)
        bbase = wid * bpw

        def slot(ref, b):
            return ref.at[pl.ds(b * 128, 128), :]

        def start(c, b):
            pltpu.async_copy(am_hbm.at[idxa_v.at[c]], slot(bufa_v, b),
                             sema[b])
            pltpu.async_copy(msg_hbm.at[idxb_v.at[c]], slot(bufb_v, b),
                             semb[b])

        def wait_in(b):
            pltpu.make_async_copy(
                am_hbm.at[idxa_v.at[0]], slot(bufa_v, b), sema[b]).wait()
            pltpu.make_async_copy(
                msg_hbm.at[idxb_v.at[0]], slot(bufb_v, b), semb[b]).wait()

        def wait_out(b):
            pltpu.make_async_copy(
                slot(outb_v, b), out_hbm.at[pl.ds(bbase, 128)],
                semo[b]).wait()

        for b in range(nbuf):
            start(b, b)

        def outer(j, carry):
            cc = j * nbuf
            for b in range(nbuf):
                c = cc + b

                @pl.when(cc > 0)
                def _drain():
                    wait_out(b)

                wait_in(b)

                def sub_r(r, carry2):
                    for g in range(H // LANES):
                        sl = pl.ds(g * LANES, LANES)
                        outb_v[b * 128 + r, sl] = (
                            bufa_v[b * 128 + r, sl] - bufb_v[b * 128 + r, sl])
                    return carry2

                lax.fori_loop(0, 128, sub_r, 0)
                pltpu.async_copy(
                    slot(outb_v, b), out_hbm.at[pl.ds(bbase + c * 128, 128)],
                    semo[b])

                @pl.when(c + nbuf < nch)
                def _next():
                    start(c + nbuf, b)
            return carry

        lax.fori_loop(0, nch // nbuf, outer, 0)
        for b in range(nbuf):
            wait_out(b)

    return g2


def _m1(f_bonds, W_i):
    n, fd = f_bonds.shape
    blk = 512
    grid = n // blk

    def body(x_ref, w_ref, inp_ref, msg_ref):
        y = jnp.dot(x_ref[...], w_ref[...], preferred_element_type=jnp.float32)
        inp_ref[...] = y
        msg_ref[...] = jnp.maximum(y, 0.0)

    return pl.pallas_call(
        body,
        grid=(grid,),
        in_specs=[pl.BlockSpec((blk, fd), lambda i: (i, 0)),
                  pl.BlockSpec((fd, H), lambda i: (0, 0))],
        out_specs=[pl.BlockSpec((blk, H), lambda i: (i, 0)),
                   pl.BlockSpec((blk, H), lambda i: (i, 0))],
        out_shape=[jax.ShapeDtypeStruct((n, H), jnp.float32),
                   jax.ShapeDtypeStruct((n, H), jnp.float32)],
    )(f_bonds, W_i)


def _m3(inp, pre, W_h):
    n = inp.shape[0]
    blk = 512
    grid = n // blk

    def body(i_ref, p_ref, w_ref, o_ref):
        y = i_ref[...] + jnp.dot(p_ref[...], w_ref[...],
                                 preferred_element_type=jnp.float32)
        o_ref[...] = jnp.maximum(y, 0.0)

    return pl.pallas_call(
        body,
        grid=(grid,),
        in_specs=[pl.BlockSpec((blk, H), lambda i: (i, 0)),
                  pl.BlockSpec((blk, H), lambda i: (i, 0)),
                  pl.BlockSpec((H, H), lambda i: (0, 0))],
        out_specs=pl.BlockSpec((blk, H), lambda i: (i, 0)),
        out_shape=jax.ShapeDtypeStruct((n, H), jnp.float32),
    )(inp, pre, W_h)


def _m4(f_atoms_p, am_p, mol3d, W_o, b_o2d, n_mols):
    atoms_p, afd = f_atoms_p.shape
    blk = 512
    grid = atoms_p // blk

    def body(fa_ref, am_ref, id_ref, w_ref, b_ref, out_ref, cnt_ref):
        i = pl.program_id(0)

        @pl.when(i == 0)
        def _init():
            out_ref[...] = jnp.zeros_like(out_ref)
            cnt_ref[...] = jnp.zeros_like(cnt_ref)

        hid = (jnp.dot(fa_ref[...], w_ref[:afd, :],
                       preferred_element_type=jnp.float32)
               + jnp.dot(am_ref[...], w_ref[afd:, :],
                         preferred_element_type=jnp.float32)
               + b_ref[...])
        hid = jnp.maximum(hid, 0.0)
        ids = id_ref[0, 0, :]
        onehot = (ids[:, None]
                  == lax.broadcasted_iota(jnp.int32, (blk, n_mols), 1)
                  ).astype(jnp.float32)
        out_ref[...] += lax.dot_general(
            onehot, hid, (((0,), (0,)), ((), ())),
            preferred_element_type=jnp.float32)
        cnt_ref[...] = cnt_ref[...] + jnp.sum(onehot, axis=0)[:, None]

        @pl.when(i == grid - 1)
        def _fini():
            out_ref[...] = out_ref[...] / jnp.maximum(cnt_ref[...], 1.0)

    return pl.pallas_call(
        body,
        grid=(grid,),
        in_specs=[pl.BlockSpec((blk, afd), lambda i: (i, 0)),
                  pl.BlockSpec((blk, H), lambda i: (i, 0)),
                  pl.BlockSpec((1, 1, blk), lambda i: (i, 0, 0)),
                  pl.BlockSpec((afd + H, H), lambda i: (0, 0)),
                  pl.BlockSpec((1, H), lambda i: (0, 0))],
        out_specs=pl.BlockSpec((n_mols, H), lambda i: (0, 0)),
        out_shape=jax.ShapeDtypeStruct((n_mols, H), jnp.float32),
        scratch_shapes=[pltpu.VMEM((n_mols, H), jnp.float32)],
    )(f_atoms_p, am_p, mol3d, W_o, b_o2d)


def _encode(f_atoms, f_bonds, a2b, b2a, b2revb, mol_ids,
            W_i, W_h, W_o, b_o2d, depth, n_mols):
    n_atoms, nb = a2b.shape
    n_bonds = f_bonds.shape[0]
    atoms_p = _round_up(n_atoms, 2560)
    bonds_p = _round_up(n_bonds, 32768)

    apad = jnp.arange((atoms_p - n_atoms) * nb, dtype=jnp.int32) % n_bonds
    a2b_flat = jnp.concatenate([a2b.astype(jnp.int32).reshape(-1), apad])
    bpad = jnp.arange(bonds_p - n_bonds, dtype=jnp.int32)
    b2a2d = jnp.concatenate(
        [b2a.astype(jnp.int32), bpad % n_atoms]).reshape(-1, 128)
    b2revb2d = jnp.concatenate(
        [b2revb.astype(jnp.int32), bpad % n_bonds]).reshape(-1, 128)
    f_atoms_p = jnp.pad(f_atoms, ((0, atoms_p - n_atoms), (0, 0)))
    mol3d = jnp.pad(mol_ids.astype(jnp.int32), (0, atoms_p - n_atoms),
                    constant_values=n_mols).reshape(atoms_p // 512, 1, 512)

    g1 = _make_g1(n_bonds, atoms_p, nb)
    g2 = _make_g2(bonds_p, atoms_p)

    inp, msg = _m1(f_bonds, W_i)
    for _ in range(depth - 1):
        am = g1(msg, a2b_flat)
        pre = g2(am, msg, b2a2d, b2revb2d)
        msg = _m3(inp, pre, W_h)
    am = g1(msg, a2b_flat)
    return _m4(f_atoms_p, am, mol3d, W_o, b_o2d, n_mols)


def kernel(f_atoms, f_bonds, a2b, b2a, b2revb, atom_mol_ids,
           ano_f_atoms, ano_f_bonds, ano_a2b, ano_b2a, ano_b2revb,
           ano_atom_mol_ids, W_i, W_h, W_o, b_o):
    depth = 3
    n_mols = 256
    b_o2d = b_o.reshape(1, H)
    mol_vecs = _encode(f_atoms, f_bonds, a2b, b2a, b2revb, atom_mol_ids,
                       W_i, W_h, W_o, b_o2d, depth, n_mols)
    ano_mol_vecs = _encode(ano_f_atoms, ano_f_bonds, ano_a2b, ano_b2a,
                           ano_b2revb, ano_atom_mol_ids,
                           W_i, W_h, W_o, b_o2d, depth, n_mols)
    return (mol_vecs, ano_mol_vecs)


# speedup vs baseline: 2.0663x; 1.0018x over previous
import functools

import jax
import jax.numpy as jnp
from jax import lax
from jax.experimental import pallas as pl
from jax.experimental.pallas import tpu as pltpu
from jax.experimental.pallas import tpu_sc as plsc

H = 128
NW = 32
LANES = 16


def _round_up(x, m):
    return -(-x // m) * m


def _make_g1(n_bonds, atoms_p, nb):
    apw = atoms_p // NW
    ck = 256
    ca = ck // nb
    nch = apw // ca
    idx_pw = apw * nb
    nbuf = 3
    nvisit = -(-nch // nbuf) * nbuf

    @functools.partial(
        pl.kernel,
        mesh=plsc.VectorSubcoreMesh(core_axis_name="c", subcore_axis_name="s"),
        out_type=jax.ShapeDtypeStruct((atoms_p, H), jnp.float32),
        scratch_types=[
            pltpu.VMEM((idx_pw,), jnp.int32),
            pltpu.VMEM((nbuf * ck, H), jnp.float32),
            pltpu.VMEM((nbuf * ca, H), jnp.float32),
            pltpu.SemaphoreType.DMA,
            pltpu.SemaphoreType.DMA,
            pltpu.SemaphoreType.DMA,
            pltpu.SemaphoreType.DMA,
            pltpu.SemaphoreType.DMA,
            pltpu.SemaphoreType.DMA,
        ],
    )
    def g1(msg_hbm, a2b_hbm, out_hbm, idx_v, rows_v, acc_v, *sems):
        semg = sems[0:nbuf]
        semo = sems[nbuf:2 * nbuf]
        wid = lax.axis_index("s") * 2 + lax.axis_index("c")
        pltpu.sync_copy(a2b_hbm.at[pl.ds(wid * idx_pw, idx_pw)], idx_v)
        abase = wid * apw

        def rows_slot(b):
            return rows_v.at[pl.ds(b * ck, ck), :]

        def acc_slot(b):
            return acc_v.at[pl.ds(b * ca, ca), :]

        def start(c, b):
            pltpu.async_copy(
                msg_hbm.at[idx_v.at[pl.ds(c * ck, ck)]], rows_slot(b),
                semg[b])

        def wait_in(b):
            pltpu.make_async_copy(
                msg_hbm.at[idx_v.at[pl.ds(0, ck)]], rows_slot(b),
                semg[b]).wait()

        def wait_out(b):
            pltpu.make_async_copy(
                acc_slot(b), out_hbm.at[pl.ds(abase, ca)], semo[b]).wait()

        for b in range(nbuf):
            start(b, b)

        def outer(j, carry):
            cc = j * nbuf
            for b in range(nbuf):
                c = cc + b

                @pl.when(c < nch)
                def _visit():
                    @pl.when(cc > 0)
                    def _drain():
                        wait_out(b)

                    wait_in(b)

                    def acc_a(a, carry2):
                        base = b * ck + a * nb
                        for g in range(H // LANES):
                            sl = pl.ds(g * LANES, LANES)
                            v = jnp.maximum(rows_v[base, sl], 0.0)
                            for k in range(1, nb):
                                v = v + jnp.maximum(rows_v[base + k, sl], 0.0)
                            acc_v[b * ca + a, sl] = v
                        return carry2

                    lax.fori_loop(0, ca, acc_a, 0)
                    pltpu.async_copy(
                        acc_slot(b), out_hbm.at[pl.ds(abase + c * ca, ca)],
                        semo[b])

                    @pl.when(c + nbuf < nch)
                    def _next():
                        start(c + nbuf, b)
            return carry

        lax.fori_loop(0, nvisit // nbuf, outer, 0)
        for b in range(nbuf):
            wait_out(b)

    return g1


def _make_g2(bonds_p, atoms_p):
    bpw = bonds_p // NW
    nch = bpw // 128
    rows_pw = bonds_p // 128 // NW
    nbuf = 3
    nvisit = -(-nch // nbuf) * nbuf
    assert rows_pw == nch

    @functools.partial(
        pl.kernel,
        mesh=plsc.VectorSubcoreMesh(core_axis_name="c", subcore_axis_name="s"),
        out_type=jax.ShapeDtypeStruct((bonds_p, H), jnp.float32),
        scratch_types=[
            pltpu.VMEM((rows_pw, 128), jnp.int32),
            pltpu.VMEM((rows_pw, 128), jnp.int32),
            pltpu.VMEM((nbuf * 128, H), jnp.float32),
            pltpu.VMEM((nbuf * 128, H), jnp.float32),
            pltpu.SemaphoreType.DMA,
            pltpu.SemaphoreType.DMA,
            pltpu.SemaphoreType.DMA,
            pltpu.SemaphoreType.DMA,
            pltpu.SemaphoreType.DMA,
            pltpu.SemaphoreType.DMA,
            pltpu.SemaphoreType.DMA,
            pltpu.SemaphoreType.DMA,
            pltpu.SemaphoreType.DMA,
        ],
    )
    def g2(am_hbm, msg_hbm, b2a_hbm, b2revb_hbm, out_hbm,
           idxa_v, idxb_v, bufa_v, bufb_v, *sems):
        sema = sems[0:nbuf]
        semb = sems[nbuf:2 * nbuf]
        semo = sems[2 * nbuf:3 * nbuf]
        wid = lax.axis_index("s") * 2 + lax.axis_index("c")
        pltpu.sync_copy(b2a_hbm.at[pl.ds(wid * rows_pw, rows_pw)], idxa_v)
        pltpu.sync_copy(b2revb_hbm.at[pl.ds(wid * rows_pw, rows_pw)], idxb_v)
        bbase = wid * bpw

        def slot(ref, b):
            return ref.at[pl.ds(b * 128, 128), :]

        def start(c, b):
            pltpu.async_copy(am_hbm.at[idxa_v.at[c]], slot(bufa_v, b),
                             sema[b])
            pltpu.async_copy(msg_hbm.at[idxb_v.at[c]], slot(bufb_v, b),
                             semb[b])

        def wait_in(b):
            pltpu.make_async_copy(
                am_hbm.at[idxa_v.at[0]], slot(bufa_v, b), sema[b]).wait()
            pltpu.make_async_copy(
                msg_hbm.at[idxb_v.at[0]], slot(bufb_v, b), semb[b]).wait()

        def wait_out(b):
            pltpu.make_async_copy(
                slot(bufa_v, b), out_hbm.at[pl.ds(bbase, 128)],
                semo[b]).wait()

        for b in range(nbuf):
            start(b, b)

        def outer(j, carry):
            cc = j * nbuf
            for b in range(nbuf):
                c = cc + b

                @pl.when(c < nch)
                def _visit():
                    wait_in(b)

                    def sub_r(r, carry2):
                        for g in range(H // LANES):
                            sl = pl.ds(g * LANES, LANES)
                            bufa_v[b * 128 + r, sl] = (
                                bufa_v[b * 128 + r, sl]
                                - jnp.maximum(bufb_v[b * 128 + r, sl], 0.0))
                        return carry2

                    lax.fori_loop(0, 128, sub_r, 0)
                    pltpu.async_copy(
                        slot(bufa_v, b),
                        out_hbm.at[pl.ds(bbase + c * 128, 128)], semo[b])

            for b in range(nbuf):
                c = cc + b

                @pl.when(c < nch)
                def _reissue():
                    wait_out(b)

                    @pl.when(c + nbuf < nch)
                    def _next():
                        start(c + nbuf, b)
            return carry

        lax.fori_loop(0, nvisit // nbuf, outer, 0)

    return g2


def _m1(f_bonds, W_i):
    n, fd = f_bonds.shape
    blk = 512
    grid = n // blk

    def body(x_ref, w_ref, inp_ref):
        inp_ref[...] = jnp.dot(x_ref[...], w_ref[...],
                               preferred_element_type=jnp.float32)

    return pl.pallas_call(
        body,
        grid=(grid,),
        in_specs=[pl.BlockSpec((blk, fd), lambda i: (i, 0)),
                  pl.BlockSpec((fd, H), lambda i: (0, 0))],
        out_specs=pl.BlockSpec((blk, H), lambda i: (i, 0)),
        out_shape=jax.ShapeDtypeStruct((n, H), jnp.float32),
    )(f_bonds, W_i)


def _m3(inp, pre, W_h):
    n = inp.shape[0]
    blk = 512
    grid = n // blk

    def body(i_ref, p_ref, w_ref, o_ref):
        o_ref[...] = i_ref[...] + jnp.dot(p_ref[...], w_ref[...],
                                          preferred_element_type=jnp.float32)

    return pl.pallas_call(
        body,
        grid=(grid,),
        in_specs=[pl.BlockSpec((blk, H), lambda i: (i, 0)),
                  pl.BlockSpec((blk, H), lambda i: (i, 0)),
                  pl.BlockSpec((H, H), lambda i: (0, 0))],
        out_specs=pl.BlockSpec((blk, H), lambda i: (i, 0)),
        out_shape=jax.ShapeDtypeStruct((n, H), jnp.float32),
    )(inp, pre, W_h)


def _m4(f_atoms_p, am_p, mol3d, W_o, b_o2d, n_mols):
    atoms_p, afd = f_atoms_p.shape
    blk = 512
    grid = atoms_p // blk

    def body(fa_ref, am_ref, id_ref, w_ref, b_ref, out_ref, cnt_ref):
        i = pl.program_id(0)

        @pl.when(i == 0)
        def _init():
            out_ref[...] = jnp.zeros_like(out_ref)
            cnt_ref[...] = jnp.zeros_like(cnt_ref)

        hid = (jnp.dot(fa_ref[...], w_ref[:afd, :],
                       preferred_element_type=jnp.float32)
               + jnp.dot(am_ref[...], w_ref[afd:, :],
                         preferred_element_type=jnp.float32)
               + b_ref[...])
        hid = jnp.maximum(hid, 0.0)
        ids = id_ref[0, 0, :]
        onehot = (ids[:, None]
                  == lax.broadcasted_iota(jnp.int32, (blk, n_mols), 1)
                  ).astype(jnp.float32)
        out_ref[...] += lax.dot_general(
            onehot, hid, (((0,), (0,)), ((), ())),
            preferred_element_type=jnp.float32)
        cnt_ref[...] = cnt_ref[...] + jnp.sum(onehot, axis=0)[:, None]

        @pl.when(i == grid - 1)
        def _fini():
            out_ref[...] = out_ref[...] / jnp.maximum(cnt_ref[...], 1.0)

    return pl.pallas_call(
        body,
        grid=(grid,),
        in_specs=[pl.BlockSpec((blk, afd), lambda i: (i, 0)),
                  pl.BlockSpec((blk, H), lambda i: (i, 0)),
                  pl.BlockSpec((1, 1, blk), lambda i: (i, 0, 0)),
                  pl.BlockSpec((afd + H, H), lambda i: (0, 0)),
                  pl.BlockSpec((1, H), lambda i: (0, 0))],
        out_specs=pl.BlockSpec((n_mols, H), lambda i: (0, 0)),
        out_shape=jax.ShapeDtypeStruct((n_mols, H), jnp.float32),
        scratch_shapes=[pltpu.VMEM((n_mols, H), jnp.float32)],
    )(f_atoms_p, am_p, mol3d, W_o, b_o2d)


def _encode(f_atoms, f_bonds, a2b, b2a, b2revb, mol_ids,
            W_i, W_h, W_o, b_o2d, depth, n_mols):
    n_atoms, nb = a2b.shape
    n_bonds = f_bonds.shape[0]
    atoms_p = _round_up(n_atoms, 2560)
    bonds_p = _round_up(n_bonds, 32768)

    apad = jnp.arange((atoms_p - n_atoms) * nb, dtype=jnp.int32) % n_bonds
    a2b_flat = jnp.concatenate([a2b.astype(jnp.int32).reshape(-1), apad])
    bpad = jnp.arange(bonds_p - n_bonds, dtype=jnp.int32)
    b2a2d = jnp.concatenate(
        [b2a.astype(jnp.int32), bpad % n_atoms]).reshape(-1, 128)
    b2revb2d = jnp.concatenate(
        [b2revb.astype(jnp.int32), bpad % n_bonds]).reshape(-1, 128)
    f_atoms_p = jnp.pad(f_atoms, ((0, atoms_p - n_atoms), (0, 0)))
    mol3d = jnp.pad(mol_ids.astype(jnp.int32), (0, atoms_p - n_atoms),
                    constant_values=n_mols).reshape(atoms_p // 512, 1, 512)

    g1 = _make_g1(n_bonds, atoms_p, nb)
    g2 = _make_g2(bonds_p, atoms_p)

    inp = _m1(f_bonds, W_i)
    msg = inp
    for _ in range(depth - 1):
        am = g1(msg, a2b_flat)
        pre = g2(am, msg, b2a2d, b2revb2d)
        msg = _m3(inp, pre, W_h)
    am = g1(msg, a2b_flat)
    return _m4(f_atoms_p, am, mol3d, W_o, b_o2d, n_mols)


def kernel(f_atoms, f_bonds, a2b, b2a, b2revb, atom_mol_ids,
           ano_f_atoms, ano_f_bonds, ano_a2b, ano_b2a, ano_b2revb,
           ano_atom_mol_ids, W_i, W_h, W_o, b_o):
    depth = 3
    n_mols = 256
    b_o2d = b_o.reshape(1, H)
    mol_vecs = _encode(f_atoms, f_bonds, a2b, b2a, b2revb, atom_mol_ids,
                       W_i, W_h, W_o, b_o2d, depth, n_mols)
    ano_mol_vecs = _encode(ano_f_atoms, ano_f_bonds, ano_a2b, ano_b2a,
                           ano_b2revb, ano_atom_mol_ids,
                           W_i, W_h, W_o, b_o2d, depth, n_mols)
    return (mol_vecs, ano_mol_vecs)
